# Initial kernel scaffold; baseline (speedup 1.0000x reference)
import functools

import jax
import jax.numpy as jnp
from jax import lax
from jax.experimental import pallas as pl
from jax.experimental.pallas import tpu as pltpu
from jax.experimental.pallas import tpu_sc as plsc

N_NODES = 50000
EMB = 64
N_USERS = 25000
HALF = 25000

NC = 2
NS = 16
LANES = 16

E_PAD = 819200
EPS = E_PAD // NS
B = 512
CH = 128
NB = EPS // B
ACC_ROWS = HALF + 8
TRASH = HALF


def _layer_body(x_hbm, rows_hbm, cols_hbm, vals_hbm, out_hbm,
                acc, rows_v, cols_v, vals_v, gath_v, gsem):
    cid = lax.axis_index("c")
    sid = lax.axis_index("s")
    dst_base = cid * HALF

    zero = jnp.zeros((LANES,), jnp.float32)

    def _zrow(j, carry):
        for f in range(EMB // LANES):
            gath_v[j, pl.ds(LANES * f, LANES)] = zero
        return carry

    lax.fori_loop(0, B, _zrow, 0)
    zbase = sid * (ACC_ROWS // NS)
    pltpu.sync_copy(gath_v.at[pl.ds(0, B)], acc.at[pl.ds(zbase, B)])
    pltpu.sync_copy(gath_v.at[pl.ds(0, 1563 - B)],
                    acc.at[pl.ds(zbase + B, 1563 - B)])
    plsc.subcore_barrier()

    def _batch(b, carry):
        r128 = sid * (EPS // CH) + b * (B // CH)
        off = sid * EPS + b * B
        pltpu.sync_copy(rows_hbm.at[pl.ds(r128, B // CH)], rows_v)
        pltpu.sync_copy(cols_hbm.at[pl.ds(r128, B // CH)], cols_v)
        pltpu.sync_copy(vals_hbm.at[pl.ds(off, B)], vals_v)

        copies = []
        for i in range(B // CH):
            copies.append(pltpu.async_copy(
                x_hbm.at[cols_v.at[i]],
                gath_v.at[pl.ds(i * CH, CH)], gsem))
        for c in copies:
            c.wait()

        for i in range(B // CH):
            def _lrow(k, carry, i=i):
                r = rows_v[i, pl.ds(k * LANES, LANES)]
                li = r - dst_base
                ok = (li >= 0) & (li < HALF)
                rows_v[i, pl.ds(k * LANES, LANES)] = jnp.where(ok, li, TRASH)
                return carry
            lax.fori_loop(0, CH // LANES, _lrow, 0)

        def _scale(j, carry):
            v = plsc.load_gather(vals_v, [jnp.full((LANES,), j, jnp.int32)])
            for f in range(EMB // LANES):
                g = gath_v[j, pl.ds(LANES * f, LANES)]
                gath_v[j, pl.ds(LANES * f, LANES)] = g * v
            return carry

        lax.fori_loop(0, B, _scale, 0)

        for i in range(B // CH):
            pltpu.sync_copy(gath_v.at[pl.ds(i * CH, CH)],
                            acc.at[rows_v.at[i]], add=True)
        return carry

    lax.fori_loop(0, NB, _batch, 0)
    plsc.subcore_barrier()

    wb = sid * (HALF // NS)
    pltpu.sync_copy(acc.at[pl.ds(wb, HALF // NS)],
                    out_hbm.at[pl.ds(dst_base + wb, HALF // NS)])

    @pl.when(sid == 0)
    def _tail():
        pltpu.sync_copy(acc.at[pl.ds((HALF // NS) * NS, HALF % NS)],
                        out_hbm.at[pl.ds(dst_base + (HALF // NS) * NS,
                                         HALF % NS)])


def _layer(x, rows2, cols2, vals):
    mesh = plsc.VectorSubcoreMesh(core_axis_name="c", subcore_axis_name="s")
    return pl.kernel(
        _layer_body,
        out_type=jax.ShapeDtypeStruct((N_NODES, EMB), jnp.float32),
        mesh=mesh,
        scratch_types=[
            pltpu.VMEM_SHARED((ACC_ROWS, EMB), jnp.float32),
            pltpu.VMEM((B // CH, CH), jnp.int32),
            pltpu.VMEM((B // CH, CH), jnp.int32),
            pltpu.VMEM((B,), jnp.float32),
            pltpu.VMEM((B, EMB), jnp.float32),
            pltpu.SemaphoreType.DMA,
        ],
    )(x, rows2, cols2, vals)


def _mean_body(x1_ref, x2_ref, x3_ref, o_ref):
    o_ref[...] = (x1_ref[...] + x2_ref[...] + x3_ref[...]) * (1.0 / 3.0)


def _mean(x1, x2, x3):
    blk = 1000
    return pl.pallas_call(
        _mean_body,
        grid=(N_NODES // blk,),
        in_specs=[pl.BlockSpec((blk, EMB), lambda i: (i, 0))] * 3,
        out_specs=pl.BlockSpec((blk, EMB), lambda i: (i, 0)),
        out_shape=jax.ShapeDtypeStruct((N_NODES, EMB), jnp.float32),
    )(x1, x2, x3)


def kernel(ego_embeddings, adj_indices, adj_values):
    rows = adj_indices[0].astype(jnp.int32)
    cols = adj_indices[1].astype(jnp.int32)
    vals = adj_values.astype(jnp.float32)
    pad = E_PAD - rows.shape[0]
    rows2 = jnp.pad(rows, (0, pad)).reshape(E_PAD // CH, CH)
    cols2 = jnp.pad(cols, (0, pad)).reshape(E_PAD // CH, CH)
    vals_p = jnp.pad(vals, (0, pad))

    x1 = _layer(ego_embeddings, rows2, cols2, vals_p)
    x2 = _layer(x1, rows2, cols2, vals_p)
    x3 = _layer(x2, rows2, cols2, vals_p)
    mean = _mean(x1, x2, x3)
    return (mean[:N_USERS], mean[N_USERS:])


# speedup vs baseline: 1.9216x; 1.9216x over previous
import functools

import jax
import jax.numpy as jnp
from jax import lax
from jax.experimental import pallas as pl
from jax.experimental.pallas import tpu as pltpu
from jax.experimental.pallas import tpu_sc as plsc

N_NODES = 50000
EMB = 64
N_USERS = 25000
HALF = 25000

NC = 2
NS = 16
LANES = 16

E_PAD = 819200
EPS = E_PAD // NS
B = 256
CH = 128
NB = EPS // B
WB = 1568
ACC_ROWS = WB * NS
TRASH = HALF


def _layer_body(x_hbm, rows_hbm, cols_hbm, vals_hbm, out_hbm,
                acc, rows_v, cols_v, vals_v, gath_v, gsem):
    cid = lax.axis_index("c")
    sid = lax.axis_index("s")
    dst_base = cid * HALF

    zero = jnp.zeros((LANES,), jnp.float32)

    def _zrow(j, carry):
        for f in range(EMB // LANES):
            gath_v[j, pl.ds(LANES * f, LANES)] = zero
        return carry

    lax.fori_loop(0, B, _zrow, 0)
    zbase = sid * WB
    for z in range(WB // B):
        pltpu.sync_copy(gath_v.at[pl.ds(0, B)],
                        acc.at[pl.ds(zbase + z * B, B)])
    pltpu.sync_copy(gath_v.at[pl.ds(0, WB % B)],
                    acc.at[pl.ds(zbase + (WB // B) * B, WB % B)])
    plsc.subcore_barrier()

    def _batch(b, carry):
        r128 = sid * (EPS // CH) + b * (B // CH)
        off = sid * EPS + b * B
        pltpu.sync_copy(rows_hbm.at[pl.ds(r128, B // CH)], rows_v)
        pltpu.sync_copy(cols_hbm.at[pl.ds(r128, B // CH)], cols_v)
        pltpu.sync_copy(vals_hbm.at[pl.ds(off, B)], vals_v)

        copies = []
        for i in range(B // CH):
            copies.append(pltpu.async_copy(
                x_hbm.at[cols_v.at[i]],
                gath_v.at[pl.ds(i * CH, CH)], gsem))
        for c in copies:
            c.wait()

        for i in range(B // CH):
            def _lrow(k, carry, i=i):
                r = rows_v[i, pl.ds(k * LANES, LANES)]
                li = r - dst_base
                ok = (li >= 0) & (li < HALF)
                rows_v[i, pl.ds(k * LANES, LANES)] = jnp.where(ok, li, TRASH)
                return carry
            lax.fori_loop(0, CH // LANES, _lrow, 0)

        def _scale(jj, carry):
            vv = vals_v[pl.ds(jj * LANES, LANES)]
            for m in range(LANES):
                v = vv.at[jnp.full((LANES,), m, jnp.int32)].get(
                    mode="promise_in_bounds")
                j = jj * LANES + m
                for f in range(EMB // LANES):
                    g = gath_v[j, pl.ds(LANES * f, LANES)]
                    gath_v[j, pl.ds(LANES * f, LANES)] = g * v
            return carry

        lax.fori_loop(0, B // LANES, _scale, 0)

        for i in range(B // CH):
            pltpu.sync_copy(gath_v.at[pl.ds(i * CH, CH)],
                            acc.at[rows_v.at[i]], add=True)
        return carry

    lax.fori_loop(0, NB, _batch, 0)
    plsc.subcore_barrier()

    wb = sid * WB

    @pl.when(sid < NS - 1)
    def _full():
        pltpu.sync_copy(acc.at[pl.ds(wb, WB)],
                        out_hbm.at[pl.ds(dst_base + wb, WB)])

    @pl.when(sid == NS - 1)
    def _tail():
        pltpu.sync_copy(acc.at[pl.ds((NS - 1) * WB, HALF - (NS - 1) * WB)],
                        out_hbm.at[pl.ds(dst_base + (NS - 1) * WB,
                                         HALF - (NS - 1) * WB)])


def _layer(x, rows2, cols2, vals):
    mesh = plsc.VectorSubcoreMesh(core_axis_name="c", subcore_axis_name="s")
    return pl.kernel(
        _layer_body,
        out_type=jax.ShapeDtypeStruct((N_NODES, EMB), jnp.float32),
        mesh=mesh,
        compiler_params=pltpu.CompilerParams(use_tc_tiling_on_sc=False),
        scratch_types=[
            pltpu.VMEM_SHARED((ACC_ROWS, EMB), jnp.float32),
            pltpu.VMEM((B // CH, CH), jnp.int32),
            pltpu.VMEM((B // CH, CH), jnp.int32),
            pltpu.VMEM((B,), jnp.float32),
            pltpu.VMEM((B, EMB), jnp.float32),
            pltpu.SemaphoreType.DMA,
        ],
    )(x, rows2, cols2, vals)


def _mean_body(x1_ref, x2_ref, x3_ref, o_ref):
    o_ref[...] = (x1_ref[...] + x2_ref[...] + x3_ref[...]) * (1.0 / 3.0)


def _mean(x1, x2, x3):
    blk = 1000
    return pl.pallas_call(
        _mean_body,
        grid=(N_NODES // blk,),
        in_specs=[pl.BlockSpec((blk, EMB), lambda i: (i, 0))] * 3,
        out_specs=pl.BlockSpec((blk, EMB), lambda i: (i, 0)),
        out_shape=jax.ShapeDtypeStruct((N_NODES, EMB), jnp.float32),
    )(x1, x2, x3)


def kernel(ego_embeddings, adj_indices, adj_values):
    rows = adj_indices[0].astype(jnp.int32)
    cols = adj_indices[1].astype(jnp.int32)
    vals = adj_values.astype(jnp.float32)
    pad = E_PAD - rows.shape[0]
    rows2 = jnp.pad(rows, (0, pad)).reshape(E_PAD // CH, CH)
    cols2 = jnp.pad(cols, (0, pad)).reshape(E_PAD // CH, CH)
    vals_p = jnp.pad(vals, (0, pad))

    x1 = _layer(ego_embeddings, rows2, cols2, vals_p)
    x2 = _layer(x1, rows2, cols2, vals_p)
    x3 = _layer(x2, rows2, cols2, vals_p)
    mean = _mean(x1, x2, x3)
    return (mean[:N_USERS], mean[N_USERS:])


# speedup vs baseline: 2.7606x; 1.4366x over previous
import jax
import jax.numpy as jnp
from jax import lax
from jax.experimental import pallas as pl
from jax.experimental.pallas import tpu as pltpu
from jax.experimental.pallas import tpu_sc as plsc

N_NODES = 50000
EMB = 64
N_USERS = 25000
HALF = 25000

NC = 2
NS = 16
LANES = 16

E_PAD = 819200
EPS = E_PAD // NS
B = 128
NB = EPS // B
WB = 1568
ACC_ROWS = WB * NS
TRASH = HALF


def _scale_rows(gath, meta):
    def _scale(jj, carry):
        vv = lax.bitcast_convert_type(meta[2, pl.ds(jj * LANES, LANES)],
                                      jnp.float32)
        for m in range(LANES):
            v = vv.at[jnp.full((LANES,), m, jnp.int32)].get(
                mode="promise_in_bounds")
            j = jj * LANES + m
            for f in range(EMB // LANES):
                g = gath[j, pl.ds(LANES * f, LANES)]
                gath[j, pl.ds(LANES * f, LANES)] = g * v
        return carry

    lax.fori_loop(0, B // LANES, _scale, 0)


def _layer_body(x_hbm, meta_hbm, out_hbm, acc,
                meta0, meta1, srows0, srows1, gath0, gath1,
                isem0, isem1, gsem0, gsem1, ssem0, ssem1):
    cid = lax.axis_index("c")
    sid = lax.axis_index("s")
    dst_base = cid * HALF
    meta = (meta0, meta1)
    srows = (srows0, srows1)
    gath = (gath0, gath1)
    isem = (isem0, isem1)
    gsem = (gsem0, gsem1)
    ssem = (ssem0, ssem1)

    zero = jnp.zeros((LANES,), jnp.float32)

    def _zrow(j, carry):
        for f in range(EMB // LANES):
            gath0[j, pl.ds(LANES * f, LANES)] = zero
        return carry

    lax.fori_loop(0, B, _zrow, 0)
    zbase = sid * WB
    for z in range(WB // B):
        pltpu.sync_copy(gath0.at[pl.ds(0, B)], acc.at[pl.ds(zbase + z * B, B)])
    pltpu.sync_copy(gath0.at[pl.ds(0, WB % B)],
                    acc.at[pl.ds(zbase + (WB // B) * B, WB % B)])
    plsc.subcore_barrier()

    cbase = sid * NB

    pltpu.sync_copy(meta_hbm.at[cbase], meta0)
    pltpu.async_copy(x_hbm.at[meta0.at[1]], gath0, gsem0)
    pltpu.async_copy(meta_hbm.at[cbase + 1], meta1, isem1)

    def _batch(gb, par):
        opar = 1 - par
        pltpu.make_async_copy(x_hbm.at[meta[par].at[1]], gath[par],
                              gsem[par]).wait()

        def _lrow(k, carry):
            r = meta[par][0, pl.ds(k * LANES, LANES)]
            li = r - dst_base
            ok = (li >= 0) & (li < HALF)
            srows[par][pl.ds(k * LANES, LANES)] = jnp.where(ok, li, TRASH)
            return carry

        lax.fori_loop(0, B // LANES, _lrow, 0)
        _scale_rows(gath[par], meta[par])

        @pl.when(gb + 1 < NB)
        def _next_gather():
            @pl.when(gb >= 1)
            def _drain_prev_scatter():
                pltpu.make_async_copy(gath[opar], acc.at[srows[opar]],
                                      ssem[opar]).wait()

            pltpu.make_async_copy(meta_hbm.at[cbase + gb + 1], meta[opar],
                                  isem[opar]).wait()
            pltpu.async_copy(x_hbm.at[meta[opar].at[1]], gath[opar],
                             gsem[opar])

        pltpu.async_copy(gath[par], acc.at[srows[par]], ssem[par], add=True)

        @pl.when(gb + 2 < NB)
        def _next_meta():
            pltpu.async_copy(meta_hbm.at[cbase + gb + 2], meta[par],
                             isem[par])

    def _pair(g, carry):
        _batch(2 * g, 0)
        _batch(2 * g + 1, 1)
        return carry

    lax.fori_loop(0, NB // 2, _pair, 0)

    pltpu.make_async_copy(gath0, acc.at[srows0], ssem0).wait()
    pltpu.make_async_copy(gath1, acc.at[srows1], ssem1).wait()
    plsc.subcore_barrier()

    wb = sid * WB

    @pl.when(sid < NS - 1)
    def _full():
        pltpu.sync_copy(acc.at[pl.ds(wb, WB)],
                        out_hbm.at[pl.ds(dst_base + wb, WB)])

    @pl.when(sid == NS - 1)
    def _tail():
        pltpu.sync_copy(acc.at[pl.ds((NS - 1) * WB, HALF - (NS - 1) * WB)],
                        out_hbm.at[pl.ds(dst_base + (NS - 1) * WB,
                                         HALF - (NS - 1) * WB)])


def _layer(x, meta):
    mesh = plsc.VectorSubcoreMesh(core_axis_name="c", subcore_axis_name="s")
    return pl.kernel(
        _layer_body,
        out_type=jax.ShapeDtypeStruct((N_NODES, EMB), jnp.float32),
        mesh=mesh,
        compiler_params=pltpu.CompilerParams(use_tc_tiling_on_sc=False),
        scratch_types=[
            pltpu.VMEM_SHARED((ACC_ROWS, EMB), jnp.float32),
            pltpu.VMEM((3, B), jnp.int32),
            pltpu.VMEM((3, B), jnp.int32),
            pltpu.VMEM((B,), jnp.int32),
            pltpu.VMEM((B,), jnp.int32),
            pltpu.VMEM((B, EMB), jnp.float32),
            pltpu.VMEM((B, EMB), jnp.float32),
            pltpu.SemaphoreType.DMA,
            pltpu.SemaphoreType.DMA,
            pltpu.SemaphoreType.DMA,
            pltpu.SemaphoreType.DMA,
            pltpu.SemaphoreType.DMA,
            pltpu.SemaphoreType.DMA,
        ],
    )(x, meta)


def _mean_body(x1_ref, x2_ref, x3_ref, o_ref):
    o_ref[...] = (x1_ref[...] + x2_ref[...] + x3_ref[...]) * (1.0 / 3.0)


def _mean(x1, x2, x3):
    blk = 1000
    return pl.pallas_call(
        _mean_body,
        grid=(N_NODES // blk,),
        in_specs=[pl.BlockSpec((blk, EMB), lambda i: (i, 0))] * 3,
        out_specs=pl.BlockSpec((blk, EMB), lambda i: (i, 0)),
        out_shape=jax.ShapeDtypeStruct((N_NODES, EMB), jnp.float32),
    )(x1, x2, x3)


def kernel(ego_embeddings, adj_indices, adj_values):
    rows = adj_indices[0].astype(jnp.int32)
    cols = adj_indices[1].astype(jnp.int32)
    vals = adj_values.astype(jnp.float32)
    pad = E_PAD - rows.shape[0]
    rows2 = jnp.pad(rows, (0, pad)).reshape(E_PAD // B, B)
    cols2 = jnp.pad(cols, (0, pad)).reshape(E_PAD // B, B)
    vbits = lax.bitcast_convert_type(jnp.pad(vals, (0, pad)),
                                     jnp.int32).reshape(E_PAD // B, B)
    meta = jnp.stack([rows2, cols2, vbits], axis=1)

    x1 = _layer(ego_embeddings, meta)
    x2 = _layer(x1, meta)
    x3 = _layer(x2, meta)
    mean = _mean(x1, x2, x3)
    return (mean[:N_USERS], mean[N_USERS:])
